# baseline (speedup 1.0000x reference)
import functools

import jax
import jax.numpy as jnp
from jax import lax
from jax.experimental import pallas as pl
from jax.experimental.pallas import tpu as pltpu
from jax.experimental.pallas import tpu_sc as plsc

ROWS = 128
COLS = 100000
CHUNK = 50000
CHUNKS_PER_ROW = COLS // CHUNK
NUM_TILES = 32
ROWS_PER_TILE = ROWS // NUM_TILES
LANES = 16
VECS_PER_CHUNK = CHUNK // LANES

NACC = 5
GROUPS = 5
BODY = NACC * GROUPS
STEPS = VECS_PER_CHUNK // BODY

_BIG_I32 = 2**31 - 1


def _scan_chunk(buf, chunk_vec_base, accs):
    def body(k, accs):
        accs = list(accs)
        for u in range(NACC):
            v = buf[pl.ds((k + u) * LANES, LANES)]
            s = jnp.broadcast_to(chunk_vec_base + k + u, (LANES,))
            bv, bs = accs[u]
            m = v > bv
            accs[u] = (jnp.where(m, v, bv), jnp.where(m, s, bs))
        return tuple(accs)

    return plsc.parallel_loop(
        0, VECS_PER_CHUNK, step=NACC, unroll=GROUPS, carry=tuple(accs)
    )(body)


@functools.partial(
    pl.kernel,
    out_type=jax.ShapeDtypeStruct((NUM_TILES * LANES,), jnp.float32),
    mesh=plsc.VectorSubcoreMesh(core_axis_name="c", subcore_axis_name="s"),
    scratch_types=[
        pltpu.VMEM((CHUNK,), jnp.float32),
        pltpu.VMEM((CHUNK,), jnp.float32),
        pltpu.VMEM((LANES,), jnp.float32),
        pltpu.SemaphoreType.DMA,
        pltpu.SemaphoreType.DMA,
    ],
    compiler_params=pltpu.CompilerParams(needs_layout_passes=False),
)
def _argmax_sc(logits_hbm, out_hbm, buf0, buf1, res_v, sem0, sem1):
    wid = lax.axis_index("s") * 2 + lax.axis_index("c")
    row0 = wid * ROWS_PER_TILE
    bufs = (buf0, buf1)
    sems = (sem0, sem1)

    n_chunks = ROWS_PER_TILE * CHUNKS_PER_ROW

    def start(t):
        r = t // CHUNKS_PER_ROW
        c = t % CHUNKS_PER_ROW
        off = (row0 + r) * COLS + c * CHUNK
        return pltpu.async_copy(
            logits_hbm.at[pl.ds(off, CHUNK)],
            bufs[t % 2],
            sems[t % 2],
        )

    def fresh_accs():
        return [
            (
                jnp.full((LANES,), -jnp.inf, jnp.float32),
                jnp.zeros((LANES,), jnp.int32),
            )
            for _ in range(NACC)
        ]

    copies = [None] * n_chunks
    copies[0] = start(0)

    lane = lax.iota(jnp.int32, LANES)
    res = jnp.zeros((LANES,), jnp.float32)
    accs = fresh_accs()
    for t in range(n_chunks):
        if t + 1 < n_chunks:
            copies[t + 1] = start(t + 1)
        copies[t].wait()
        c = t % CHUNKS_PER_ROW
        accs = _scan_chunk(bufs[t % 2], jnp.int32(c * VECS_PER_CHUNK), accs)
        if c == CHUNKS_PER_ROW - 1:
            bv, bi = accs[0][0], accs[0][1] * LANES + lane
            for u in range(1, NACC):
                v2, i2 = accs[u][0], accs[u][1] * LANES + lane
                better = (v2 > bv) | ((v2 == bv) & (i2 < bi))
                bv = jnp.where(better, v2, bv)
                bi = jnp.where(better, i2, bi)
            m = jnp.max(bv)
            cand = jnp.where(bv == m, bi, jnp.int32(_BIG_I32))
            win = jnp.min(cand).astype(jnp.float32)
            r = t // CHUNKS_PER_ROW
            res = jnp.where(lane == r, win, res)
            accs = fresh_accs()

    res_v[...] = res
    pltpu.sync_copy(res_v, out_hbm.at[pl.ds(wid * LANES, LANES)])


def kernel(logits):
    flat = logits.reshape(ROWS * COLS)
    out = _argmax_sc(flat)
    return out.reshape(NUM_TILES, LANES)[:, :ROWS_PER_TILE].reshape(ROWS)


# speedup vs baseline: 1.5674x; 1.5674x over previous
import functools

import jax
import jax.numpy as jnp
from jax import lax
from jax.experimental import pallas as pl
from jax.experimental.pallas import tpu as pltpu
from jax.experimental.pallas import tpu_sc as plsc

ROWS = 128
COLS = 100000
LANES = 16
TILE_COLS = 128

CHUNK_TILES = 30
CHUNK_COLS = CHUNK_TILES * TILE_COLS
N_CHUNKS = 13
HALF_TILES = CHUNK_TILES * N_CHUNKS
EPI_COL = 2 * HALF_TILES * TILE_COLS
EPI_COLS = COLS - EPI_COL

_BIG_I32 = 2**31 - 1


def _scan_chunk(buf, ncols, colbase, accs):
    def body(v, accs):
        accs = list(accs)
        s = jnp.broadcast_to(colbase + v * LANES, (LANES,))
        for r in range(8):
            x = buf[r, pl.ds(v * LANES, LANES)]
            bv, bs = accs[r]
            m = x > bv
            accs[r] = (jnp.where(m, x, bv), jnp.where(m, s, bs))
        return tuple(accs)

    return plsc.parallel_loop(
        0, ncols // LANES, step=1, unroll=4, carry=tuple(accs)
    )(body)


@functools.partial(
    pl.kernel,
    out_type=jax.ShapeDtypeStruct((256,), jnp.float32),
    mesh=plsc.VectorSubcoreMesh(core_axis_name="c", subcore_axis_name="s"),
    scratch_types=[
        pltpu.VMEM((8, CHUNK_COLS), jnp.float32),
        pltpu.VMEM((8, CHUNK_COLS), jnp.float32),
        pltpu.VMEM((8, EPI_COLS), jnp.float32),
        pltpu.VMEM((LANES,), jnp.float32),
        pltpu.VMEM((LANES,), jnp.int32),
        pltpu.VMEM((LANES,), jnp.float32),
        pltpu.VMEM((LANES,), jnp.int32),
        pltpu.VMEM((LANES,), jnp.float32),
        pltpu.VMEM_SHARED((16, LANES), jnp.float32),
        pltpu.VMEM_SHARED((16, LANES), jnp.int32),
        pltpu.SemaphoreType.DMA,
        pltpu.SemaphoreType.DMA,
        pltpu.SemaphoreType.DMA,
    ],
    compiler_params=pltpu.CompilerParams(needs_layout_passes=False),
)
def _argmax_sc(
    logits_hbm, out_hbm,
    buf0, buf1, ebuf,
    stage_v, stage_i, part_v, part_i, res_v,
    sval, sidx,
    sem0, sem1, seme,
):
    c = lax.axis_index("c")
    s = lax.axis_index("s")
    rg = c * 8 + lax.rem(s, 8)
    h = s // 8
    row0 = pl.multiple_of(rg * 8, 8)
    bufs = (buf0, buf1)
    sems = (sem0, sem1)

    def start(k):
        cb = pl.multiple_of((h * HALF_TILES + k * CHUNK_TILES) * TILE_COLS,
                            TILE_COLS)
        copy = pltpu.async_copy(
            logits_hbm.at[pl.ds(row0, 8), pl.ds(cb, CHUNK_COLS)],
            bufs[k % 2],
            sems[k % 2],
        )
        return copy, cb

    epi_copy = pltpu.async_copy(
        logits_hbm.at[pl.ds(row0, 8), pl.ds(EPI_COL, EPI_COLS)], ebuf, seme
    )

    copies = [None] * N_CHUNKS
    cbs = [None] * N_CHUNKS
    copies[0], cbs[0] = start(0)

    accs = tuple(
        (
            jnp.full((LANES,), -jnp.inf, jnp.float32),
            jnp.zeros((LANES,), jnp.int32),
        )
        for _ in range(8)
    )
    for k in range(N_CHUNKS):
        if k + 1 < N_CHUNKS:
            copies[k + 1], cbs[k + 1] = start(k + 1)
        copies[k].wait()
        accs = _scan_chunk(bufs[k % 2], CHUNK_COLS, cbs[k], accs)

    epi_copy.wait()
    accs = _scan_chunk(ebuf, EPI_COLS, jnp.int32(EPI_COL), accs)

    lane = lax.iota(jnp.int32, LANES)
    valp = jnp.full((LANES,), -jnp.inf, jnp.float32)
    idxp = jnp.zeros((LANES,), jnp.int32)
    for r in range(8):
        bv, bs = accs[r]
        idx = bs + lane
        m = jnp.max(bv)
        cand = jnp.where(bv == m, idx, jnp.int32(_BIG_I32))
        win = jnp.min(cand)
        valp = jnp.where(lane == r, m, valp)
        idxp = jnp.where(lane == r, win, idxp)

    stage_v[...] = valp
    stage_i[...] = idxp
    pltpu.sync_copy(stage_v, sval.at[s])
    pltpu.sync_copy(stage_i, sidx.at[s])
    plsc.subcore_barrier()

    @pl.when(h == 0)
    def _merge_and_write():
        pltpu.sync_copy(sval.at[s + 8], part_v)
        pltpu.sync_copy(sidx.at[s + 8], part_i)
        v2 = part_v[...]
        i2 = part_i[...]
        better = (v2 > valp) | ((v2 == valp) & (i2 < idxp))
        fin = jnp.where(better, i2, idxp)
        res_v[...] = fin.astype(jnp.float32)
        w2 = c * 8 + s
        pltpu.sync_copy(res_v, out_hbm.at[pl.ds(w2 * LANES, LANES)])


def kernel(logits):
    out = _argmax_sc(logits)
    return out.reshape(16, LANES)[:, :8].reshape(ROWS)
